# baseline (speedup 1.0000x reference)
import jax
import jax.numpy as jnp
from jax import lax
from jax.experimental import pallas as pl

_K_TOP = 32
_BISECT_ITERS = 16


def _body(emb_ref, xbin_ref, wbinT_ref, wcat_ref, w0b0_ref, out_ref):
    f32 = jnp.float32
    BB = out_ref.shape[0]
    BF = xbin_ref.shape[1]

    emb = emb_ref[:, 1:, :]
    ls = jnp.mean(emb, axis=-1)
    agg = lax.dot_general(
        ls, wbinT_ref[...], (((1,), (0,)), ((), ())),
        preferred_element_type=f32,
    )

    lo = jnp.min(agg, axis=1, keepdims=True)
    hi = jnp.max(agg, axis=1, keepdims=True)

    def bisect(_, carry):
        lo, hi = carry
        mid = 0.5 * (lo + hi)
        cnt = jnp.sum((agg >= mid).astype(f32), axis=1, keepdims=True)
        pred = cnt >= _K_TOP
        return jnp.where(pred, mid, lo), jnp.where(pred, hi, mid)

    lo, _ = lax.fori_loop(0, _BISECT_ITERS, bisect, (lo, hi))
    big = jnp.float32(3.4e38)
    thr0 = jnp.min(jnp.where(agg >= lo, agg, big), axis=1, keepdims=True)
    above = agg > thr0
    cs = jnp.sum(above.astype(f32), axis=1, keepdims=True)
    thr1 = jnp.min(jnp.where(above, agg, big), axis=1, keepdims=True)
    thr = jnp.where(cs >= _K_TOP, thr1, thr0)

    mask = (agg >= thr).astype(f32)
    mxb = mask * xbin_ref[...]

    G = lax.dot_general(
        mxb, wcat_ref[...], (((1,), (0,)), ((), ())),
        preferred_element_type=f32,
    )

    emb0 = emb_ref[:, 0, :]
    w0 = w0b0_ref[0:1, 0:128]
    b0 = w0b0_ref[0:1, 128:129]
    dot0 = jnp.sum(emb0 * (G[:, 0:128] + w0), axis=1, keepdims=True)
    out_ref[...] = dot0 + G[:, 128:129] + b0


def kernel(x_t, x_m_emb, x_bin, W_bin, W_reg, b_reg):
    B, Fp1, D = x_m_emb.shape
    BF = x_bin.shape[1]
    BB = 256
    grid = (B // BB,)

    wbinT = W_bin.T
    wcat = jnp.concatenate([W_reg[1:, :], b_reg[1:, None]], axis=1)
    w0b0 = jnp.concatenate([W_reg[0:1, :], b_reg[0:1, None]], axis=1)

    out = pl.pallas_call(
        _body,
        grid=grid,
        in_specs=[
            pl.BlockSpec((BB, Fp1, D), lambda i: (i, 0, 0)),
            pl.BlockSpec((BB, BF), lambda i: (i, 0)),
            pl.BlockSpec((Fp1 - 1, BF), lambda i: (0, 0)),
            pl.BlockSpec((BF, D + 1), lambda i: (0, 0)),
            pl.BlockSpec((1, D + 1), lambda i: (0, 0)),
        ],
        out_specs=pl.BlockSpec((BB, 1), lambda i: (i, 0)),
        out_shape=jax.ShapeDtypeStruct((B, 1), jnp.float32),
    )(x_m_emb, x_bin, wbinT, wcat, w0b0)
    return out


# speedup vs baseline: 1.0958x; 1.0958x over previous
import jax
import jax.numpy as jnp
from jax import lax
from jax.experimental import pallas as pl

_K_TOP = 32
_BISECT_ITERS = 16


def _body(emb_ref, xbin_ref, wbinT_ref, wcat_ref, w0b0_ref, out_ref):
    f32 = jnp.float32
    BB = out_ref.shape[0]
    BF = xbin_ref.shape[1]

    emb = emb_ref[:, 1:, :]
    ls = jnp.mean(emb, axis=-1)
    agg = lax.dot_general(
        ls, wbinT_ref[...], (((1,), (0,)), ((), ())),
        preferred_element_type=f32,
    )

    lo = jnp.min(agg, axis=1, keepdims=True)
    hi = jnp.max(agg, axis=1, keepdims=True)

    def bisect(_, carry):
        lo, hi = carry
        mid = 0.5 * (lo + hi)
        cnt = jnp.sum((agg >= mid).astype(f32), axis=1, keepdims=True)
        pred = cnt >= _K_TOP
        return jnp.where(pred, mid, lo), jnp.where(pred, hi, mid)

    lo, _ = lax.fori_loop(0, _BISECT_ITERS, bisect, (lo, hi))
    big = jnp.float32(3.4e38)
    thr0 = jnp.min(jnp.where(agg >= lo, agg, big), axis=1, keepdims=True)
    above = agg > thr0
    cs = jnp.sum(above.astype(f32), axis=1, keepdims=True)
    thr1 = jnp.min(jnp.where(above, agg, big), axis=1, keepdims=True)
    thr = jnp.where(cs >= _K_TOP, thr1, thr0)

    mask = (agg >= thr).astype(f32)
    mxb = mask * xbin_ref[...]

    G = lax.dot_general(
        mxb, wcat_ref[...], (((1,), (0,)), ((), ())),
        preferred_element_type=f32,
    )

    emb0 = emb_ref[:, 0, :]
    w0 = w0b0_ref[0:1, 0:128]
    b0 = w0b0_ref[0:1, 128:129]
    dot0 = jnp.sum(emb0 * (G[:, 0:128] + w0), axis=1, keepdims=True)
    out_ref[...] = dot0 + G[:, 128:129] + b0


def kernel(x_t, x_m_emb, x_bin, W_bin, W_reg, b_reg):
    B, Fp1, D = x_m_emb.shape
    BF = x_bin.shape[1]
    BB = 512
    grid = (B // BB,)

    wbinT = W_bin.T
    wcat = jnp.concatenate([W_reg[1:, :], b_reg[1:, None]], axis=1)
    w0b0 = jnp.concatenate([W_reg[0:1, :], b_reg[0:1, None]], axis=1)

    out = pl.pallas_call(
        _body,
        grid=grid,
        in_specs=[
            pl.BlockSpec((BB, Fp1, D), lambda i: (i, 0, 0)),
            pl.BlockSpec((BB, BF), lambda i: (i, 0)),
            pl.BlockSpec((Fp1 - 1, BF), lambda i: (0, 0)),
            pl.BlockSpec((BF, D + 1), lambda i: (0, 0)),
            pl.BlockSpec((1, D + 1), lambda i: (0, 0)),
        ],
        out_specs=pl.BlockSpec((BB, 1), lambda i: (i, 0)),
        out_shape=jax.ShapeDtypeStruct((B, 1), jnp.float32),
    )(x_m_emb, x_bin, wbinT, wcat, w0b0)
    return out


# speedup vs baseline: 1.0964x; 1.0005x over previous
import jax
import jax.numpy as jnp
from jax import lax
from jax.experimental import pallas as pl

_K_TOP = 32
_BISECT_ITERS = 16


def _body(emb_ref, xbin_ref, wbinT_ref, wcat_ref, w0b0_ref, out_ref):
    f32 = jnp.float32
    BB = out_ref.shape[0]
    BF = xbin_ref.shape[1]

    emb = emb_ref[:, 1:, :]
    ls = jnp.mean(emb, axis=-1)
    agg = lax.dot_general(
        ls, wbinT_ref[...], (((1,), (0,)), ((), ())),
        preferred_element_type=f32,
    )

    half = BB // 2
    aggA = agg[:half]
    aggB = agg[half:]

    def init(a):
        return (jnp.min(a, axis=1, keepdims=True),
                jnp.max(a, axis=1, keepdims=True))

    loA, hiA = init(aggA)
    loB, hiB = init(aggB)

    def step(a, lo, hi):
        mid = 0.5 * (lo + hi)
        cnt = jnp.sum((a >= mid).astype(f32), axis=1, keepdims=True)
        pred = cnt >= _K_TOP
        return jnp.where(pred, mid, lo), jnp.where(pred, hi, mid)

    def bisect(_, carry):
        loA, hiA, loB, hiB = carry
        loA, hiA = step(aggA, loA, hiA)
        loB, hiB = step(aggB, loB, hiB)
        return loA, hiA, loB, hiB

    loA, _, loB, _ = lax.fori_loop(
        0, _BISECT_ITERS, bisect, (loA, hiA, loB, hiB))
    lo = jnp.concatenate([loA, loB], axis=0)
    big = jnp.float32(3.4e38)
    thr0 = jnp.min(jnp.where(agg >= lo, agg, big), axis=1, keepdims=True)
    above = agg > thr0
    cs = jnp.sum(above.astype(f32), axis=1, keepdims=True)
    thr1 = jnp.min(jnp.where(above, agg, big), axis=1, keepdims=True)
    thr = jnp.where(cs >= _K_TOP, thr1, thr0)

    mask = (agg >= thr).astype(f32)
    mxb = mask * xbin_ref[...]

    G = lax.dot_general(
        mxb, wcat_ref[...], (((1,), (0,)), ((), ())),
        preferred_element_type=f32,
    )

    emb0 = emb_ref[:, 0, :]
    w0 = w0b0_ref[0:1, 0:128]
    b0 = w0b0_ref[0:1, 128:129]
    dot0 = jnp.sum(emb0 * (G[:, 0:128] + w0), axis=1, keepdims=True)
    out_ref[...] = dot0 + G[:, 128:129] + b0


def kernel(x_t, x_m_emb, x_bin, W_bin, W_reg, b_reg):
    B, Fp1, D = x_m_emb.shape
    BF = x_bin.shape[1]
    BB = 512
    grid = (B // BB,)

    wbinT = W_bin.T
    wcat = jnp.concatenate([W_reg[1:, :], b_reg[1:, None]], axis=1)
    w0b0 = jnp.concatenate([W_reg[0:1, :], b_reg[0:1, None]], axis=1)

    out = pl.pallas_call(
        _body,
        grid=grid,
        in_specs=[
            pl.BlockSpec((BB, Fp1, D), lambda i: (i, 0, 0)),
            pl.BlockSpec((BB, BF), lambda i: (i, 0)),
            pl.BlockSpec((Fp1 - 1, BF), lambda i: (0, 0)),
            pl.BlockSpec((BF, D + 1), lambda i: (0, 0)),
            pl.BlockSpec((1, D + 1), lambda i: (0, 0)),
        ],
        out_specs=pl.BlockSpec((BB, 1), lambda i: (i, 0)),
        out_shape=jax.ShapeDtypeStruct((B, 1), jnp.float32),
    )(x_m_emb, x_bin, wbinT, wcat, w0b0)
    return out
